# Initial kernel scaffold; baseline (speedup 1.0000x reference)
import jax
import jax.numpy as jnp
from jax.experimental import pallas as pl


def kernel(t1, t2, H_indices, H_values, W1, b1, W2, b2, v):
    raise NotImplementedError("write your pallas kernel here")


# speedup vs baseline: 165.4738x; 165.4738x over previous
import functools

import jax
import jax.numpy as jnp
from jax import lax
from jax.experimental import pallas as pl
from jax.experimental.pallas import tpu as pltpu
from jax.experimental.pallas import tpu_sc as plsc

B, N1, N2 = 4, 2048, 2048
F1 = F2 = 256
H_DIM = 256
NNZ = 262144

NSEG = B * N1
NW = 32
CHUNK = NNZ // NW
VECS = CHUNK // 16
STRIPE = NSEG // 16
MV_BLK = 512


def _mv_body(t1_ref, t2_ref, w1_ref, w2_ref, v_ref, a1_ref, a2_ref):
    u1 = lax.dot_general(v_ref[...], w1_ref[...], (((1,), (0,)), ((), ())),
                         preferred_element_type=jnp.float32)
    u2 = lax.dot_general(v_ref[...], w2_ref[...], (((1,), (0,)), ((), ())),
                         preferred_element_type=jnp.float32)
    a1_ref[...] = lax.dot_general(t1_ref[...], u1, (((1,), (1,)), ((), ())),
                                  preferred_element_type=jnp.float32)
    a2_ref[...] = lax.dot_general(t2_ref[...], u2, (((1,), (1,)), ((), ())),
                                  preferred_element_type=jnp.float32)


_matvec = pl.pallas_call(
    _mv_body,
    grid=(B * N1 // MV_BLK,),
    in_specs=[
        pl.BlockSpec((MV_BLK, F1), lambda i: (i, 0)),
        pl.BlockSpec((MV_BLK, F2), lambda i: (i, 0)),
        pl.BlockSpec((H_DIM, F1), lambda i: (0, 0)),
        pl.BlockSpec((H_DIM, F2), lambda i: (0, 0)),
        pl.BlockSpec((1, H_DIM), lambda i: (0, 0)),
    ],
    out_specs=[pl.BlockSpec((MV_BLK, 1), lambda i: (i, 0)),
               pl.BlockSpec((MV_BLK, 1), lambda i: (i, 0))],
    out_shape=[jax.ShapeDtypeStruct((B * N1, 1), jnp.float32),
               jax.ShapeDtypeStruct((B * N2, 1), jnp.float32)],
)

_MESH = plsc.VectorSubcoreMesh(core_axis_name="c", subcore_axis_name="s")
_SC_PARAMS = pltpu.CompilerParams(needs_layout_passes=False)


@functools.partial(
    pl.kernel,
    out_type=(jax.ShapeDtypeStruct((NNZ,), jnp.float32),
              jax.ShapeDtypeStruct((2, NSEG), jnp.float32)),
    mesh=_MESH,
    compiler_params=_SC_PARAMS,
    scratch_types=[
        pltpu.VMEM((NSEG,), jnp.float32),
        pltpu.VMEM((NSEG,), jnp.float32),
        pltpu.VMEM((CHUNK,), jnp.int32),
        pltpu.VMEM((CHUNK,), jnp.int32),
        pltpu.VMEM((CHUNK,), jnp.float32),
        pltpu.VMEM((NSEG,), jnp.float32),
        pltpu.VMEM((STRIPE,), jnp.float32),
        pltpu.VMEM((STRIPE,), jnp.float32),
        pltpu.VMEM_SHARED((16, NSEG), jnp.float32),
    ],
)
def _sc_logits(a1_hbm, a2_hbm, seg_hbm, col_hbm, ew_hbm, ps_hbm,
               a1_v, a2_v, seg_v, col_v, ew_v, tbl_v, acc_v, tmp_v, shared):
    cid = lax.axis_index("c")
    sid = lax.axis_index("s")
    base = (cid * 16 + sid) * CHUNK
    pltpu.sync_copy(a1_hbm, a1_v)
    pltpu.sync_copy(a2_hbm, a2_v)
    pltpu.sync_copy(seg_hbm.at[pl.ds(base, CHUNK)], seg_v)
    pltpu.sync_copy(col_hbm.at[pl.ds(base, CHUNK)], col_v)

    zeros16 = jnp.zeros((16,), jnp.float32)

    def zero_body(i, carry):
        tbl_v[pl.ds(i * 16, 16)] = zeros16
        return carry

    lax.fori_loop(0, NSEG // 16, zero_body, 0)

    def entry_body(i, carry):
        off = i * 16
        seg = seg_v[pl.ds(off, 16)]
        col = col_v[pl.ds(off, 16)]
        e1 = plsc.load_gather(a1_v, [seg])
        e2 = plsc.load_gather(a2_v, [col])
        e = jnp.exp(e1 + e2)
        ew_v[pl.ds(off, 16)] = e
        plsc.addupdate_scatter(tbl_v, [seg], e)
        return carry

    lax.fori_loop(0, VECS, entry_body, 0)

    pltpu.sync_copy(ew_v, ew_hbm.at[pl.ds(base, CHUNK)])

    pltpu.sync_copy(tbl_v, shared.at[sid])
    plsc.subcore_barrier()
    rbase = sid * STRIPE
    pltpu.sync_copy(shared.at[0, pl.ds(rbase, STRIPE)], acc_v)
    for k in range(1, 16):
        pltpu.sync_copy(shared.at[k, pl.ds(rbase, STRIPE)], tmp_v)

        def add_body(j, carry):
            sl = pl.ds(j * 16, 16)
            acc_v[sl] = acc_v[sl] + tmp_v[sl]
            return carry

        lax.fori_loop(0, STRIPE // 16, add_body, 0)
    pltpu.sync_copy(acc_v, ps_hbm.at[cid, pl.ds(rbase, STRIPE)])


@functools.partial(
    pl.kernel,
    out_type=jax.ShapeDtypeStruct((NNZ,), jnp.float32),
    mesh=_MESH,
    compiler_params=_SC_PARAMS,
    scratch_types=[
        pltpu.VMEM((NSEG,), jnp.float32),
        pltpu.VMEM((NSEG,), jnp.float32),
        pltpu.VMEM((NSEG,), jnp.float32),
        pltpu.VMEM((CHUNK,), jnp.int32),
        pltpu.VMEM((CHUNK,), jnp.float32),
        pltpu.VMEM((CHUNK,), jnp.float32),
    ],
)
def _sc_normalize(ps_hbm, seg_hbm, ew_hbm, out_hbm,
                  p0_v, p1_v, inv_v, sg_v, ew_v, o_v):
    cid = lax.axis_index("c")
    sid = lax.axis_index("s")
    base = (cid * 16 + sid) * CHUNK
    pltpu.sync_copy(ps_hbm.at[0], p0_v)
    pltpu.sync_copy(ps_hbm.at[1], p1_v)
    pltpu.sync_copy(seg_hbm.at[pl.ds(base, CHUNK)], sg_v)
    pltpu.sync_copy(ew_hbm.at[pl.ds(base, CHUNK)], ew_v)

    ones16 = jnp.ones((16,), jnp.float32)

    def inv_body(i, carry):
        sl = pl.ds(i * 16, 16)
        inv_v[sl] = ones16 / (p0_v[sl] + p1_v[sl])
        return carry

    lax.fori_loop(0, NSEG // 16, inv_body, 0)

    def entry_body(i, carry):
        off = i * 16
        seg = sg_v[pl.ds(off, 16)]
        g = plsc.load_gather(inv_v, [seg])
        o_v[pl.ds(off, 16)] = ew_v[pl.ds(off, 16)] * g
        return carry

    lax.fori_loop(0, VECS, entry_body, 0)

    pltpu.sync_copy(o_v, out_hbm.at[pl.ds(base, CHUNK)])


def kernel(t1, t2, H_indices, H_values, W1, b1, W2, b2, v):
    del H_values, b1, b2
    t1f = t1.reshape(B * N1, F1)
    t2f = t2.reshape(B * N2, F2)
    v2d = v.reshape(1, H_DIM)
    a1, a2 = _matvec(t1f, t2f, W1, W2, v2d)
    seg = H_indices[0] * N1 + H_indices[1]
    col = H_indices[0] * N2 + H_indices[2]
    ew, ps = _sc_logits(a1.reshape(NSEG), a2.reshape(NSEG), seg, col)
    return _sc_normalize(ps, seg, ew)


# speedup vs baseline: 165.8038x; 1.0020x over previous
import functools

import jax
import jax.numpy as jnp
from jax import lax
from jax.experimental import pallas as pl
from jax.experimental.pallas import tpu as pltpu
from jax.experimental.pallas import tpu_sc as plsc

B, N1, N2 = 4, 2048, 2048
F1 = F2 = 256
H_DIM = 256
NNZ = 262144

NSEG = B * N1
NW = 32
CHUNK = NNZ // NW
VECS = CHUNK // 16
STRIPE = NSEG // 16
MV_BLK = 512


def _mv_body(t1_ref, t2_ref, w1_ref, w2_ref, v_ref, a1_ref, a2_ref):
    u1 = lax.dot_general(v_ref[...], w1_ref[...], (((1,), (0,)), ((), ())),
                         preferred_element_type=jnp.float32)
    u2 = lax.dot_general(v_ref[...], w2_ref[...], (((1,), (0,)), ((), ())),
                         preferred_element_type=jnp.float32)
    a1_ref[...] = lax.dot_general(t1_ref[...], u1, (((1,), (1,)), ((), ())),
                                  preferred_element_type=jnp.float32)
    a2_ref[...] = lax.dot_general(t2_ref[...], u2, (((1,), (1,)), ((), ())),
                                  preferred_element_type=jnp.float32)


_matvec = pl.pallas_call(
    _mv_body,
    grid=(B * N1 // MV_BLK,),
    in_specs=[
        pl.BlockSpec((MV_BLK, F1), lambda i: (i, 0)),
        pl.BlockSpec((MV_BLK, F2), lambda i: (i, 0)),
        pl.BlockSpec((H_DIM, F1), lambda i: (0, 0)),
        pl.BlockSpec((H_DIM, F2), lambda i: (0, 0)),
        pl.BlockSpec((1, H_DIM), lambda i: (0, 0)),
    ],
    out_specs=[pl.BlockSpec((MV_BLK, 1), lambda i: (i, 0)),
               pl.BlockSpec((MV_BLK, 1), lambda i: (i, 0))],
    out_shape=[jax.ShapeDtypeStruct((B * N1, 1), jnp.float32),
               jax.ShapeDtypeStruct((B * N2, 1), jnp.float32)],
)

_MESH = plsc.VectorSubcoreMesh(core_axis_name="c", subcore_axis_name="s")
_SC_PARAMS = pltpu.CompilerParams(needs_layout_passes=False)


@functools.partial(
    pl.kernel,
    out_type=(jax.ShapeDtypeStruct((NNZ,), jnp.float32),
              jax.ShapeDtypeStruct((2, NSEG), jnp.float32)),
    mesh=_MESH,
    compiler_params=_SC_PARAMS,
    scratch_types=[
        pltpu.VMEM((NSEG,), jnp.float32),
        pltpu.VMEM((NSEG,), jnp.float32),
        pltpu.VMEM((CHUNK,), jnp.int32),
        pltpu.VMEM((CHUNK,), jnp.int32),
        pltpu.VMEM((CHUNK,), jnp.int32),
        pltpu.VMEM((CHUNK,), jnp.float32),
        pltpu.VMEM((NSEG,), jnp.float32),
        pltpu.VMEM((STRIPE,), jnp.float32),
        pltpu.VMEM((16 * STRIPE,), jnp.float32),
        pltpu.VMEM_SHARED((16, NSEG), jnp.float32),
        pltpu.SemaphoreType.DMA,
    ],
)
def _sc_logits(a1_hbm, a2_hbm, idx_hbm, ew_hbm, ps_hbm,
               a1_v, a2_v, b_v, r_v, c_v, ew_v, tbl_v, acc_v, tmp_v, shared,
               sem):
    cid = lax.axis_index("c")
    sid = lax.axis_index("s")
    base = (cid * 16 + sid) * CHUNK
    pltpu.sync_copy(a1_hbm, a1_v)
    pltpu.sync_copy(a2_hbm, a2_v)
    pltpu.sync_copy(idx_hbm.at[pl.ds(base, CHUNK)], b_v)
    pltpu.sync_copy(idx_hbm.at[pl.ds(NNZ + base, CHUNK)], r_v)
    pltpu.sync_copy(idx_hbm.at[pl.ds(2 * NNZ + base, CHUNK)], c_v)

    zeros16 = jnp.zeros((16,), jnp.float32)

    def zero_body(i, carry):
        tbl_v[pl.ds(i * 16, 16)] = zeros16
        return carry

    lax.fori_loop(0, NSEG // 16, zero_body, 0)

    def entry_body(i, carry):
        off = i * 16
        sl = pl.ds(off, 16)
        seg = b_v[sl] * N1 + r_v[sl]
        col = b_v[sl] * N2 + c_v[sl]
        e1 = plsc.load_gather(a1_v, [seg])
        e2 = plsc.load_gather(a2_v, [col])
        e = jnp.exp(e1 + e2)
        ew_v[sl] = e
        plsc.addupdate_scatter(tbl_v, [seg], e)
        return carry

    lax.fori_loop(0, VECS, entry_body, 0)

    pltpu.sync_copy(ew_v, ew_hbm.at[pl.ds(base, CHUNK)])

    pltpu.sync_copy(tbl_v, shared.at[sid])
    plsc.subcore_barrier()
    rbase = sid * STRIPE
    copies = [pltpu.async_copy(shared.at[k, pl.ds(rbase, STRIPE)],
                               tmp_v.at[pl.ds(k * STRIPE, STRIPE)], sem)
              for k in range(16)]
    for cp in copies:
        cp.wait()
    for j in range(STRIPE // 16):
        sl = pl.ds(j * 16, 16)
        acc16 = tmp_v[pl.ds(j * 16, 16)]
        for k in range(1, 16):
            acc16 = acc16 + tmp_v[pl.ds(k * STRIPE + j * 16, 16)]
        acc_v[sl] = acc16
    pltpu.sync_copy(acc_v, ps_hbm.at[cid, pl.ds(rbase, STRIPE)])


@functools.partial(
    pl.kernel,
    out_type=jax.ShapeDtypeStruct((NNZ,), jnp.float32),
    mesh=_MESH,
    compiler_params=_SC_PARAMS,
    scratch_types=[
        pltpu.VMEM((NSEG,), jnp.float32),
        pltpu.VMEM((NSEG,), jnp.float32),
        pltpu.VMEM((NSEG,), jnp.float32),
        pltpu.VMEM((CHUNK,), jnp.int32),
        pltpu.VMEM((CHUNK,), jnp.int32),
        pltpu.VMEM((CHUNK,), jnp.float32),
        pltpu.VMEM((CHUNK,), jnp.float32),
    ],
)
def _sc_normalize(ps_hbm, idx_hbm, ew_hbm, out_hbm,
                  p0_v, p1_v, inv_v, b_v, r_v, ew_v, o_v):
    cid = lax.axis_index("c")
    sid = lax.axis_index("s")
    base = (cid * 16 + sid) * CHUNK
    pltpu.sync_copy(ps_hbm.at[0], p0_v)
    pltpu.sync_copy(ps_hbm.at[1], p1_v)
    pltpu.sync_copy(idx_hbm.at[pl.ds(base, CHUNK)], b_v)
    pltpu.sync_copy(idx_hbm.at[pl.ds(NNZ + base, CHUNK)], r_v)
    pltpu.sync_copy(ew_hbm.at[pl.ds(base, CHUNK)], ew_v)

    ones16 = jnp.ones((16,), jnp.float32)

    def inv_body(i, carry):
        sl = pl.ds(i * 16, 16)
        inv_v[sl] = ones16 / (p0_v[sl] + p1_v[sl])
        return carry

    lax.fori_loop(0, NSEG // 16, inv_body, 0)

    def entry_body(i, carry):
        sl = pl.ds(i * 16, 16)
        seg = b_v[sl] * N1 + r_v[sl]
        g = plsc.load_gather(inv_v, [seg])
        o_v[sl] = ew_v[sl] * g
        return carry

    lax.fori_loop(0, VECS, entry_body, 0)

    pltpu.sync_copy(o_v, out_hbm.at[pl.ds(base, CHUNK)])


def kernel(t1, t2, H_indices, H_values, W1, b1, W2, b2, v):
    del H_values, b1, b2
    t1f = t1.reshape(B * N1, F1)
    t2f = t2.reshape(B * N2, F2)
    v2d = v.reshape(1, H_DIM)
    a1, a2 = _matvec(t1f, t2f, W1, W2, v2d)
    idx_flat = H_indices.reshape(3 * NNZ)
    ew, ps = _sc_logits(a1.reshape(NSEG), a2.reshape(NSEG), idx_flat)
    return _sc_normalize(ps, idx_flat, ew)


# speedup vs baseline: 200.2449x; 1.2077x over previous
import functools

import jax
import jax.numpy as jnp
from jax import lax
from jax.experimental import pallas as pl
from jax.experimental.pallas import tpu as pltpu
from jax.experimental.pallas import tpu_sc as plsc

B, N1, N2 = 4, 2048, 2048
F1 = F2 = 256
H_DIM = 256
NNZ = 262144

NSEG = B * N1
NW = 32
CHUNK = NNZ // NW
VECS = CHUNK // 16
STRIPE = NSEG // 16
MV_BLK = 512


def _mv_body(t1_ref, t2_ref, w1_ref, w2_ref, v_ref, a1_ref, a2_ref):
    u1 = lax.dot_general(v_ref[...], w1_ref[...], (((1,), (0,)), ((), ())),
                         preferred_element_type=jnp.float32)
    u2 = lax.dot_general(v_ref[...], w2_ref[...], (((1,), (0,)), ((), ())),
                         preferred_element_type=jnp.float32)
    a1_ref[...] = lax.dot_general(t1_ref[...], u1, (((1,), (1,)), ((), ())),
                                  preferred_element_type=jnp.float32)
    a2_ref[...] = lax.dot_general(t2_ref[...], u2, (((1,), (1,)), ((), ())),
                                  preferred_element_type=jnp.float32)


_matvec = pl.pallas_call(
    _mv_body,
    grid=(B * N1 // MV_BLK,),
    in_specs=[
        pl.BlockSpec((MV_BLK, F1), lambda i: (i, 0)),
        pl.BlockSpec((MV_BLK, F2), lambda i: (i, 0)),
        pl.BlockSpec((H_DIM, F1), lambda i: (0, 0)),
        pl.BlockSpec((H_DIM, F2), lambda i: (0, 0)),
        pl.BlockSpec((1, H_DIM), lambda i: (0, 0)),
    ],
    out_specs=[pl.BlockSpec((MV_BLK, 1), lambda i: (i, 0)),
               pl.BlockSpec((MV_BLK, 1), lambda i: (i, 0))],
    out_shape=[jax.ShapeDtypeStruct((B * N1, 1), jnp.float32),
               jax.ShapeDtypeStruct((B * N2, 1), jnp.float32)],
)

_MESH = plsc.VectorSubcoreMesh(core_axis_name="c", subcore_axis_name="s")
_SC_PARAMS = pltpu.CompilerParams(needs_layout_passes=False)


@functools.partial(
    pl.kernel,
    out_type=(jax.ShapeDtypeStruct((NNZ,), jnp.float32),
              jax.ShapeDtypeStruct((2, NSEG), jnp.float32)),
    mesh=_MESH,
    compiler_params=_SC_PARAMS,
    scratch_types=[
        pltpu.VMEM((NSEG,), jnp.float32),
        pltpu.VMEM((NSEG,), jnp.float32),
        pltpu.VMEM((CHUNK,), jnp.int32),
        pltpu.VMEM((CHUNK,), jnp.int32),
        pltpu.VMEM((CHUNK,), jnp.int32),
        pltpu.VMEM((CHUNK,), jnp.float32),
        pltpu.VMEM((NSEG,), jnp.float32),
        pltpu.VMEM((STRIPE,), jnp.float32),
        pltpu.VMEM((16 * STRIPE,), jnp.float32),
        pltpu.VMEM_SHARED((16, NSEG), jnp.float32),
        pltpu.SemaphoreType.DMA,
    ],
)
def _sc_logits(a1_hbm, a2_hbm, idx_hbm, ew_hbm, ps_hbm,
               a1_v, a2_v, b_v, r_v, c_v, ew_v, tbl_v, acc_v, tmp_v, shared,
               sem):
    cid = lax.axis_index("c")
    sid = lax.axis_index("s")
    base = (cid * 16 + sid) * CHUNK
    pltpu.sync_copy(a1_hbm, a1_v)
    pltpu.sync_copy(a2_hbm, a2_v)
    pltpu.sync_copy(idx_hbm.at[pl.ds(base, CHUNK)], b_v)
    pltpu.sync_copy(idx_hbm.at[pl.ds(NNZ + base, CHUNK)], r_v)
    pltpu.sync_copy(idx_hbm.at[pl.ds(2 * NNZ + base, CHUNK)], c_v)

    zeros16 = jnp.zeros((16,), jnp.float32)

    def zero_body(i, carry):
        off = i * 64
        for u in range(4):
            tbl_v[pl.ds(off + u * 16, 16)] = zeros16
        return carry

    lax.fori_loop(0, NSEG // 64, zero_body, 0)

    def entry_body(i, carry):
        off = i * 64
        sls = [pl.ds(off + u * 16, 16) for u in range(4)]
        bs = [b_v[sl] for sl in sls]
        segs = [b * N1 + r_v[sl] for b, sl in zip(bs, sls)]
        cols = [b * N2 + c_v[sl] for b, sl in zip(bs, sls)]
        e1s = [plsc.load_gather(a1_v, [seg]) for seg in segs]
        e2s = [plsc.load_gather(a2_v, [col]) for col in cols]
        es = [jnp.exp(e1 + e2) for e1, e2 in zip(e1s, e2s)]
        for sl, seg, e in zip(sls, segs, es):
            ew_v[sl] = e
            plsc.addupdate_scatter(tbl_v, [seg], e)
        return carry

    lax.fori_loop(0, VECS // 4, entry_body, 0)

    pltpu.sync_copy(ew_v, ew_hbm.at[pl.ds(base, CHUNK)])

    pltpu.sync_copy(tbl_v, shared.at[sid])
    plsc.subcore_barrier()
    rbase = sid * STRIPE
    copies = [pltpu.async_copy(shared.at[k, pl.ds(rbase, STRIPE)],
                               tmp_v.at[pl.ds(k * STRIPE, STRIPE)], sem)
              for k in range(16)]
    for cp in copies:
        cp.wait()
    for j in range(STRIPE // 16):
        sl = pl.ds(j * 16, 16)
        acc16 = tmp_v[pl.ds(j * 16, 16)]
        for k in range(1, 16):
            acc16 = acc16 + tmp_v[pl.ds(k * STRIPE + j * 16, 16)]
        acc_v[sl] = acc16
    pltpu.sync_copy(acc_v, ps_hbm.at[cid, pl.ds(rbase, STRIPE)])


@functools.partial(
    pl.kernel,
    out_type=jax.ShapeDtypeStruct((NNZ,), jnp.float32),
    mesh=_MESH,
    compiler_params=_SC_PARAMS,
    scratch_types=[
        pltpu.VMEM((STRIPE,), jnp.float32),
        pltpu.VMEM((STRIPE,), jnp.float32),
        pltpu.VMEM((STRIPE,), jnp.float32),
        pltpu.VMEM((NSEG,), jnp.float32),
        pltpu.VMEM((CHUNK,), jnp.int32),
        pltpu.VMEM((CHUNK,), jnp.int32),
        pltpu.VMEM((CHUNK,), jnp.float32),
        pltpu.VMEM((CHUNK,), jnp.float32),
        pltpu.VMEM_SHARED((NSEG,), jnp.float32),
    ],
)
def _sc_normalize(ps_hbm, idx_hbm, ew_hbm, out_hbm,
                  p0_v, p1_v, is_v, inv_v, b_v, r_v, ew_v, o_v, shared):
    cid = lax.axis_index("c")
    sid = lax.axis_index("s")
    base = (cid * 16 + sid) * CHUNK
    rbase = sid * STRIPE
    pltpu.sync_copy(ps_hbm.at[0, pl.ds(rbase, STRIPE)], p0_v)
    pltpu.sync_copy(ps_hbm.at[1, pl.ds(rbase, STRIPE)], p1_v)
    pltpu.sync_copy(idx_hbm.at[pl.ds(base, CHUNK)], b_v)
    pltpu.sync_copy(idx_hbm.at[pl.ds(NNZ + base, CHUNK)], r_v)
    pltpu.sync_copy(ew_hbm.at[pl.ds(base, CHUNK)], ew_v)

    ones16 = jnp.ones((16,), jnp.float32)
    for j in range(STRIPE // 16):
        sl = pl.ds(j * 16, 16)
        is_v[sl] = ones16 / (p0_v[sl] + p1_v[sl])
    pltpu.sync_copy(is_v, shared.at[pl.ds(rbase, STRIPE)])
    plsc.subcore_barrier()
    pltpu.sync_copy(shared, inv_v)

    def entry_body(i, carry):
        off = i * 64
        sls = [pl.ds(off + u * 16, 16) for u in range(4)]
        segs = [b_v[sl] * N1 + r_v[sl] for sl in sls]
        gs = [plsc.load_gather(inv_v, [seg]) for seg in segs]
        for sl, g in zip(sls, gs):
            o_v[sl] = ew_v[sl] * g
        return carry

    lax.fori_loop(0, VECS // 4, entry_body, 0)

    pltpu.sync_copy(o_v, out_hbm.at[pl.ds(base, CHUNK)])


def kernel(t1, t2, H_indices, H_values, W1, b1, W2, b2, v):
    del H_values, b1, b2
    t1f = t1.reshape(B * N1, F1)
    t2f = t2.reshape(B * N2, F2)
    v2d = v.reshape(1, H_DIM)
    a1, a2 = _matvec(t1f, t2f, W1, W2, v2d)
    idx_flat = H_indices.reshape(3 * NNZ)
    ew, ps = _sc_logits(a1.reshape(NSEG), a2.reshape(NSEG), idx_flat)
    return _sc_normalize(ps, idx_flat, ew)


# speedup vs baseline: 222.5926x; 1.1116x over previous
import functools

import jax
import jax.numpy as jnp
from jax import lax
from jax.experimental import pallas as pl
from jax.experimental.pallas import tpu as pltpu
from jax.experimental.pallas import tpu_sc as plsc

B, N1, N2 = 4, 2048, 2048
F1 = F2 = 256
H_DIM = 256
NNZ = 262144

NSEG = B * N1
NW = 32
CHUNK = NNZ // NW
VECS = CHUNK // 16
STRIPE = NSEG // 16
MV_BLK = 512


def _mv_body(t1_ref, t2_ref, w1_ref, w2_ref, v_ref, a1_ref, a2_ref):
    u1 = lax.dot_general(v_ref[...], w1_ref[...], (((1,), (0,)), ((), ())),
                         preferred_element_type=jnp.float32)
    u2 = lax.dot_general(v_ref[...], w2_ref[...], (((1,), (0,)), ((), ())),
                         preferred_element_type=jnp.float32)
    a1_ref[...] = lax.dot_general(t1_ref[...], u1, (((1,), (1,)), ((), ())),
                                  preferred_element_type=jnp.float32)
    a2_ref[...] = lax.dot_general(t2_ref[...], u2, (((1,), (1,)), ((), ())),
                                  preferred_element_type=jnp.float32)


_matvec = pl.pallas_call(
    _mv_body,
    grid=(B * N1 // MV_BLK,),
    in_specs=[
        pl.BlockSpec((MV_BLK, F1), lambda i: (i, 0)),
        pl.BlockSpec((MV_BLK, F2), lambda i: (i, 0)),
        pl.BlockSpec((H_DIM, F1), lambda i: (0, 0)),
        pl.BlockSpec((H_DIM, F2), lambda i: (0, 0)),
        pl.BlockSpec((1, H_DIM), lambda i: (0, 0)),
    ],
    out_specs=[pl.BlockSpec((MV_BLK, 1), lambda i: (i, 0)),
               pl.BlockSpec((MV_BLK, 1), lambda i: (i, 0))],
    out_shape=[jax.ShapeDtypeStruct((B * N1, 1), jnp.float32),
               jax.ShapeDtypeStruct((B * N2, 1), jnp.float32)],
)

_MESH = plsc.VectorSubcoreMesh(core_axis_name="c", subcore_axis_name="s")
_SC_PARAMS = pltpu.CompilerParams(needs_layout_passes=False)


@functools.partial(
    pl.kernel,
    out_type=(jax.ShapeDtypeStruct((NNZ,), jnp.float32),
              jax.ShapeDtypeStruct((2, NSEG), jnp.float32)),
    mesh=_MESH,
    compiler_params=_SC_PARAMS,
    scratch_types=[
        pltpu.VMEM((NSEG,), jnp.float32),
        pltpu.VMEM((NSEG,), jnp.float32),
        pltpu.VMEM((CHUNK,), jnp.int32),
        pltpu.VMEM((CHUNK,), jnp.int32),
        pltpu.VMEM((CHUNK,), jnp.int32),
        pltpu.VMEM((CHUNK,), jnp.float32),
        pltpu.VMEM((NSEG,), jnp.float32),
        pltpu.VMEM((STRIPE,), jnp.float32),
        pltpu.VMEM((16 * STRIPE,), jnp.float32),
        pltpu.VMEM_SHARED((16, NSEG), jnp.float32),
        pltpu.SemaphoreType.DMA,
    ],
)
def _sc_logits(a1_hbm, a2_hbm, idx_hbm, ew_hbm, ps_hbm,
               a1_v, a2_v, b_v, r_v, c_v, ew_v, tbl_v, acc_v, tmp_v, shared,
               sem):
    cid = lax.axis_index("c")
    sid = lax.axis_index("s")
    base = (cid * 16 + sid) * CHUNK
    rbase = sid * STRIPE
    copies = [
        pltpu.async_copy(a1_hbm, a1_v, sem),
        pltpu.async_copy(a2_hbm, a2_v, sem),
        pltpu.async_copy(idx_hbm.at[pl.ds(base, CHUNK)], b_v, sem),
        pltpu.async_copy(idx_hbm.at[pl.ds(NNZ + base, CHUNK)], r_v, sem),
        pltpu.async_copy(idx_hbm.at[pl.ds(2 * NNZ + base, CHUNK)], c_v, sem),
    ]

    zeros16 = jnp.zeros((16,), jnp.float32)

    def zero_body(i, carry):
        off = i * 64
        for u in range(4):
            tbl_v[pl.ds(off + u * 16, 16)] = zeros16
        return carry

    lax.fori_loop(0, NSEG // 64, zero_body, 0)
    for cp in copies:
        cp.wait()

    @plsc.parallel_loop(0, VECS, 1, unroll=4)
    def _(i):
        sl = pl.ds(i * 16, 16)
        b16 = b_v[sl]
        seg = b16 * N1 + r_v[sl]
        col = b16 * N2 + c_v[sl]
        e1 = plsc.load_gather(a1_v, [seg])
        e2 = plsc.load_gather(a2_v, [col])
        e = jnp.exp(e1 + e2)
        ew_v[sl] = e
        plsc.addupdate_scatter(tbl_v, [seg], e)

    out_cp = pltpu.async_copy(ew_v, ew_hbm.at[pl.ds(base, CHUNK)], sem)

    pltpu.sync_copy(tbl_v, shared.at[sid])
    plsc.subcore_barrier()
    red_cps = [pltpu.async_copy(shared.at[k, pl.ds(rbase, STRIPE)],
                                tmp_v.at[pl.ds(k * STRIPE, STRIPE)], sem)
               for k in range(16)]
    out_cp.wait()
    for cp in red_cps:
        cp.wait()
    for j in range(STRIPE // 16):
        sl = pl.ds(j * 16, 16)
        acc16 = tmp_v[pl.ds(j * 16, 16)]
        for k in range(1, 16):
            acc16 = acc16 + tmp_v[pl.ds(k * STRIPE + j * 16, 16)]
        acc_v[sl] = acc16
    pltpu.sync_copy(acc_v, ps_hbm.at[cid, pl.ds(rbase, STRIPE)])


@functools.partial(
    pl.kernel,
    out_type=jax.ShapeDtypeStruct((NNZ,), jnp.float32),
    mesh=_MESH,
    compiler_params=_SC_PARAMS,
    scratch_types=[
        pltpu.VMEM((STRIPE,), jnp.float32),
        pltpu.VMEM((STRIPE,), jnp.float32),
        pltpu.VMEM((STRIPE,), jnp.float32),
        pltpu.VMEM((NSEG,), jnp.float32),
        pltpu.VMEM((CHUNK,), jnp.int32),
        pltpu.VMEM((CHUNK,), jnp.int32),
        pltpu.VMEM((CHUNK,), jnp.float32),
        pltpu.VMEM((CHUNK,), jnp.float32),
        pltpu.VMEM_SHARED((NSEG,), jnp.float32),
        pltpu.SemaphoreType.DMA,
    ],
)
def _sc_normalize(ps_hbm, idx_hbm, ew_hbm, out_hbm,
                  p0_v, p1_v, is_v, inv_v, b_v, r_v, ew_v, o_v, shared, sem):
    cid = lax.axis_index("c")
    sid = lax.axis_index("s")
    base = (cid * 16 + sid) * CHUNK
    rbase = sid * STRIPE
    copies = [
        pltpu.async_copy(idx_hbm.at[pl.ds(base, CHUNK)], b_v, sem),
        pltpu.async_copy(idx_hbm.at[pl.ds(NNZ + base, CHUNK)], r_v, sem),
        pltpu.async_copy(ew_hbm.at[pl.ds(base, CHUNK)], ew_v, sem),
    ]
    pltpu.sync_copy(ps_hbm.at[0, pl.ds(rbase, STRIPE)], p0_v)
    pltpu.sync_copy(ps_hbm.at[1, pl.ds(rbase, STRIPE)], p1_v)

    ones16 = jnp.ones((16,), jnp.float32)
    for j in range(STRIPE // 16):
        sl = pl.ds(j * 16, 16)
        is_v[sl] = ones16 / (p0_v[sl] + p1_v[sl])
    pltpu.sync_copy(is_v, shared.at[pl.ds(rbase, STRIPE)])
    plsc.subcore_barrier()
    pltpu.sync_copy(shared, inv_v)
    for cp in copies:
        cp.wait()

    @plsc.parallel_loop(0, VECS, 1, unroll=4)
    def _(i):
        sl = pl.ds(i * 16, 16)
        seg = b_v[sl] * N1 + r_v[sl]
        g = plsc.load_gather(inv_v, [seg])
        o_v[sl] = ew_v[sl] * g

    pltpu.sync_copy(o_v, out_hbm.at[pl.ds(base, CHUNK)])


def kernel(t1, t2, H_indices, H_values, W1, b1, W2, b2, v):
    del H_values, b1, b2
    t1f = t1.reshape(B * N1, F1)
    t2f = t2.reshape(B * N2, F2)
    v2d = v.reshape(1, H_DIM)
    a1, a2 = _matvec(t1f, t2f, W1, W2, v2d)
    idx_flat = H_indices.reshape(3 * NNZ)
    ew, ps = _sc_logits(a1.reshape(NSEG), a2.reshape(NSEG), idx_flat)
    return _sc_normalize(ps, idx_flat, ew)


# speedup vs baseline: 251.2476x; 1.1287x over previous
import functools

import jax
import jax.numpy as jnp
from jax import lax
from jax.experimental import pallas as pl
from jax.experimental.pallas import tpu as pltpu
from jax.experimental.pallas import tpu_sc as plsc

B, N1, N2 = 4, 2048, 2048
F1 = F2 = 256
H_DIM = 256
NNZ = 262144

NSEG = B * N1
NW = 32
CHUNK = NNZ // NW
VECS = CHUNK // 16
STRIPE = NSEG // 16
MV_BLK = 2048


def _mv_body(t1_ref, t2_ref, w1_ref, w2_ref, v_ref, a1_ref, a2_ref):
    u1 = lax.dot_general(v_ref[...], w1_ref[...], (((1,), (0,)), ((), ())),
                         preferred_element_type=jnp.float32)
    u2 = lax.dot_general(v_ref[...], w2_ref[...], (((1,), (0,)), ((), ())),
                         preferred_element_type=jnp.float32)
    a1_ref[...] = lax.dot_general(t1_ref[...], u1, (((1,), (1,)), ((), ())),
                                  preferred_element_type=jnp.float32)
    a2_ref[...] = lax.dot_general(t2_ref[...], u2, (((1,), (1,)), ((), ())),
                                  preferred_element_type=jnp.float32)


_matvec = pl.pallas_call(
    _mv_body,
    grid=(B * N1 // MV_BLK,),
    in_specs=[
        pl.BlockSpec((MV_BLK, F1), lambda i: (i, 0)),
        pl.BlockSpec((MV_BLK, F2), lambda i: (i, 0)),
        pl.BlockSpec((H_DIM, F1), lambda i: (0, 0)),
        pl.BlockSpec((H_DIM, F2), lambda i: (0, 0)),
        pl.BlockSpec((1, H_DIM), lambda i: (0, 0)),
    ],
    out_specs=[pl.BlockSpec((MV_BLK, 1), lambda i: (i, 0)),
               pl.BlockSpec((MV_BLK, 1), lambda i: (i, 0))],
    out_shape=[jax.ShapeDtypeStruct((B * N1, 1), jnp.float32),
               jax.ShapeDtypeStruct((B * N2, 1), jnp.float32)],
)

_MESH = plsc.VectorSubcoreMesh(core_axis_name="c", subcore_axis_name="s")
_SC_PARAMS = pltpu.CompilerParams(needs_layout_passes=False)


@functools.partial(
    pl.kernel,
    out_type=(jax.ShapeDtypeStruct((NNZ,), jnp.float32),
              jax.ShapeDtypeStruct((2, NSEG), jnp.float32)),
    mesh=_MESH,
    compiler_params=_SC_PARAMS,
    scratch_types=[
        pltpu.VMEM((NSEG,), jnp.float32),
        pltpu.VMEM((NSEG,), jnp.float32),
        pltpu.VMEM((CHUNK,), jnp.int32),
        pltpu.VMEM((CHUNK,), jnp.int32),
        pltpu.VMEM((CHUNK,), jnp.int32),
        pltpu.VMEM((CHUNK,), jnp.float32),
        pltpu.VMEM((NSEG,), jnp.float32),
        pltpu.VMEM((STRIPE,), jnp.float32),
        pltpu.VMEM((16 * STRIPE,), jnp.float32),
        pltpu.VMEM_SHARED((16, NSEG), jnp.float32),
        pltpu.SemaphoreType.DMA,
    ],
)
def _sc_logits(a1_hbm, a2_hbm, idx_hbm, ew_hbm, ps_hbm,
               a1_v, a2_v, b_v, r_v, c_v, ew_v, tbl_v, acc_v, tmp_v, shared,
               sem):
    cid = lax.axis_index("c")
    sid = lax.axis_index("s")
    base = (cid * 16 + sid) * CHUNK
    rbase = sid * STRIPE
    copies = [
        pltpu.async_copy(a1_hbm, a1_v, sem),
        pltpu.async_copy(a2_hbm, a2_v, sem),
        pltpu.async_copy(idx_hbm.at[pl.ds(base, CHUNK)], b_v, sem),
        pltpu.async_copy(idx_hbm.at[pl.ds(NNZ + base, CHUNK)], r_v, sem),
        pltpu.async_copy(idx_hbm.at[pl.ds(2 * NNZ + base, CHUNK)], c_v, sem),
    ]

    zeros16 = jnp.zeros((16,), jnp.float32)

    def zero_body(i, carry):
        off = i * 64
        for u in range(4):
            tbl_v[pl.ds(off + u * 16, 16)] = zeros16
        return carry

    lax.fori_loop(0, NSEG // 64, zero_body, 0)
    for cp in copies:
        cp.wait()

    @plsc.parallel_loop(0, VECS, 1, unroll=4)
    def _(i):
        sl = pl.ds(i * 16, 16)
        b16 = b_v[sl]
        seg = b16 * N1 + r_v[sl]
        col = b16 * N2 + c_v[sl]
        e1 = plsc.load_gather(a1_v, [seg])
        e2 = plsc.load_gather(a2_v, [col])
        e = jnp.exp(e1 + e2)
        ew_v[sl] = e
        plsc.addupdate_scatter(tbl_v, [seg], e)

    out_cp = pltpu.async_copy(ew_v, ew_hbm.at[pl.ds(base, CHUNK)], sem)

    pltpu.sync_copy(tbl_v, shared.at[sid])
    plsc.subcore_barrier()
    red_cps = [pltpu.async_copy(shared.at[k, pl.ds(rbase, STRIPE)],
                                tmp_v.at[pl.ds(k * STRIPE, STRIPE)], sem)
               for k in range(16)]
    out_cp.wait()
    for cp in red_cps:
        cp.wait()
    for j in range(STRIPE // 16):
        sl = pl.ds(j * 16, 16)
        acc16 = tmp_v[pl.ds(j * 16, 16)]
        for k in range(1, 16):
            acc16 = acc16 + tmp_v[pl.ds(k * STRIPE + j * 16, 16)]
        acc_v[sl] = acc16
    pltpu.sync_copy(acc_v, ps_hbm.at[cid, pl.ds(rbase, STRIPE)])


@functools.partial(
    pl.kernel,
    out_type=jax.ShapeDtypeStruct((NNZ,), jnp.float32),
    mesh=_MESH,
    compiler_params=_SC_PARAMS,
    scratch_types=[
        pltpu.VMEM((STRIPE,), jnp.float32),
        pltpu.VMEM((STRIPE,), jnp.float32),
        pltpu.VMEM((STRIPE,), jnp.float32),
        pltpu.VMEM((NSEG,), jnp.float32),
        pltpu.VMEM((CHUNK,), jnp.int32),
        pltpu.VMEM((CHUNK,), jnp.int32),
        pltpu.VMEM((CHUNK,), jnp.float32),
        pltpu.VMEM((CHUNK,), jnp.float32),
        pltpu.VMEM_SHARED((NSEG,), jnp.float32),
        pltpu.SemaphoreType.DMA,
    ],
)
def _sc_normalize(ps_hbm, idx_hbm, ew_hbm, out_hbm,
                  p0_v, p1_v, is_v, inv_v, b_v, r_v, ew_v, o_v, shared, sem):
    cid = lax.axis_index("c")
    sid = lax.axis_index("s")
    base = (cid * 16 + sid) * CHUNK
    rbase = sid * STRIPE
    copies = [
        pltpu.async_copy(idx_hbm.at[pl.ds(base, CHUNK)], b_v, sem),
        pltpu.async_copy(idx_hbm.at[pl.ds(NNZ + base, CHUNK)], r_v, sem),
        pltpu.async_copy(ew_hbm.at[pl.ds(base, CHUNK)], ew_v, sem),
    ]
    pltpu.sync_copy(ps_hbm.at[0, pl.ds(rbase, STRIPE)], p0_v)
    pltpu.sync_copy(ps_hbm.at[1, pl.ds(rbase, STRIPE)], p1_v)

    ones16 = jnp.ones((16,), jnp.float32)
    for j in range(STRIPE // 16):
        sl = pl.ds(j * 16, 16)
        is_v[sl] = ones16 / (p0_v[sl] + p1_v[sl])
    pltpu.sync_copy(is_v, shared.at[pl.ds(rbase, STRIPE)])
    plsc.subcore_barrier()
    pltpu.sync_copy(shared, inv_v)
    for cp in copies:
        cp.wait()

    @plsc.parallel_loop(0, VECS, 1, unroll=4)
    def _(i):
        sl = pl.ds(i * 16, 16)
        seg = b_v[sl] * N1 + r_v[sl]
        g = plsc.load_gather(inv_v, [seg])
        o_v[sl] = ew_v[sl] * g

    pltpu.sync_copy(o_v, out_hbm.at[pl.ds(base, CHUNK)])


def kernel(t1, t2, H_indices, H_values, W1, b1, W2, b2, v):
    del H_values, b1, b2
    t1f = t1.reshape(B * N1, F1)
    t2f = t2.reshape(B * N2, F2)
    v2d = v.reshape(1, H_DIM)
    idx_flat = H_indices.reshape(3 * NNZ)
    a1, a2 = _matvec(t1f, t2f, W1, W2, v2d)
    ew, ps = _sc_logits(a1.reshape(NSEG), a2.reshape(NSEG), idx_flat)
    return _sc_normalize(ps, idx_flat, ew)


# speedup vs baseline: 253.0636x; 1.0072x over previous
import functools

import jax
import jax.numpy as jnp
from jax import lax
from jax.experimental import pallas as pl
from jax.experimental.pallas import tpu as pltpu
from jax.experimental.pallas import tpu_sc as plsc

B, N1, N2 = 4, 2048, 2048
F1 = F2 = 256
H_DIM = 256
NNZ = 262144

NSEG = B * N1
NW = 32
CHUNK = NNZ // NW
VECS = CHUNK // 16
STRIPE = NSEG // 16
MV_BLK = 4096


def _mv_body(t1_ref, t2_ref, w1_ref, w2_ref, v_ref, a1_ref, a2_ref):
    u1 = lax.dot_general(v_ref[...], w1_ref[...], (((1,), (0,)), ((), ())),
                         preferred_element_type=jnp.float32)
    u2 = lax.dot_general(v_ref[...], w2_ref[...], (((1,), (0,)), ((), ())),
                         preferred_element_type=jnp.float32)
    a1_ref[...] = lax.dot_general(t1_ref[...], u1, (((1,), (1,)), ((), ())),
                                  preferred_element_type=jnp.float32)
    a2_ref[...] = lax.dot_general(t2_ref[...], u2, (((1,), (1,)), ((), ())),
                                  preferred_element_type=jnp.float32)


_matvec = pl.pallas_call(
    _mv_body,
    grid=(B * N1 // MV_BLK,),
    in_specs=[
        pl.BlockSpec((MV_BLK, F1), lambda i: (i, 0)),
        pl.BlockSpec((MV_BLK, F2), lambda i: (i, 0)),
        pl.BlockSpec((H_DIM, F1), lambda i: (0, 0)),
        pl.BlockSpec((H_DIM, F2), lambda i: (0, 0)),
        pl.BlockSpec((1, H_DIM), lambda i: (0, 0)),
    ],
    out_specs=[pl.BlockSpec((MV_BLK, 1), lambda i: (i, 0)),
               pl.BlockSpec((MV_BLK, 1), lambda i: (i, 0))],
    out_shape=[jax.ShapeDtypeStruct((B * N1, 1), jnp.float32),
               jax.ShapeDtypeStruct((B * N2, 1), jnp.float32)],
)

_MESH = plsc.VectorSubcoreMesh(core_axis_name="c", subcore_axis_name="s")
_SC_PARAMS = pltpu.CompilerParams(needs_layout_passes=False)


@functools.partial(
    pl.kernel,
    out_type=(jax.ShapeDtypeStruct((NNZ,), jnp.float32),
              jax.ShapeDtypeStruct((2, NSEG), jnp.float32)),
    mesh=_MESH,
    compiler_params=_SC_PARAMS,
    scratch_types=[
        pltpu.VMEM((NSEG,), jnp.float32),
        pltpu.VMEM((NSEG,), jnp.float32),
        pltpu.VMEM((CHUNK,), jnp.int32),
        pltpu.VMEM((CHUNK,), jnp.int32),
        pltpu.VMEM((CHUNK,), jnp.int32),
        pltpu.VMEM((CHUNK,), jnp.float32),
        pltpu.VMEM((NSEG,), jnp.float32),
        pltpu.VMEM((STRIPE,), jnp.float32),
        pltpu.VMEM((16 * STRIPE,), jnp.float32),
        pltpu.VMEM_SHARED((16, NSEG), jnp.float32),
        pltpu.SemaphoreType.DMA,
    ],
)
def _sc_logits(a1_hbm, a2_hbm, idx_hbm, ew_hbm, ps_hbm,
               a1_v, a2_v, b_v, r_v, c_v, ew_v, tbl_v, acc_v, tmp_v, shared,
               sem):
    cid = lax.axis_index("c")
    sid = lax.axis_index("s")
    base = (cid * 16 + sid) * CHUNK
    rbase = sid * STRIPE
    copies = [
        pltpu.async_copy(a1_hbm, a1_v, sem),
        pltpu.async_copy(a2_hbm, a2_v, sem),
        pltpu.async_copy(idx_hbm.at[pl.ds(base, CHUNK)], b_v, sem),
        pltpu.async_copy(idx_hbm.at[pl.ds(NNZ + base, CHUNK)], r_v, sem),
        pltpu.async_copy(idx_hbm.at[pl.ds(2 * NNZ + base, CHUNK)], c_v, sem),
    ]

    zeros16 = jnp.zeros((16,), jnp.float32)

    def zero_body(i, carry):
        off = i * 64
        for u in range(4):
            tbl_v[pl.ds(off + u * 16, 16)] = zeros16
        return carry

    lax.fori_loop(0, NSEG // 64, zero_body, 0)
    for cp in copies:
        cp.wait()

    @plsc.parallel_loop(0, VECS, 1, unroll=4)
    def _(i):
        sl = pl.ds(i * 16, 16)
        b16 = b_v[sl]
        seg = b16 * N1 + r_v[sl]
        col = b16 * N2 + c_v[sl]
        e1 = plsc.load_gather(a1_v, [seg])
        e2 = plsc.load_gather(a2_v, [col])
        e = jnp.exp(e1 + e2)
        ew_v[sl] = e
        plsc.addupdate_scatter(tbl_v, [seg], e)

    out_cp = pltpu.async_copy(ew_v, ew_hbm.at[pl.ds(base, CHUNK)], sem)

    pltpu.sync_copy(tbl_v, shared.at[sid])
    plsc.subcore_barrier()
    red_cps = [pltpu.async_copy(shared.at[k, pl.ds(rbase, STRIPE)],
                                tmp_v.at[pl.ds(k * STRIPE, STRIPE)], sem)
               for k in range(16)]
    out_cp.wait()
    for cp in red_cps:
        cp.wait()
    for j in range(STRIPE // 16):
        sl = pl.ds(j * 16, 16)
        acc16 = tmp_v[pl.ds(j * 16, 16)]
        for k in range(1, 16):
            acc16 = acc16 + tmp_v[pl.ds(k * STRIPE + j * 16, 16)]
        acc_v[sl] = acc16
    pltpu.sync_copy(acc_v, ps_hbm.at[cid, pl.ds(rbase, STRIPE)])


@functools.partial(
    pl.kernel,
    out_type=jax.ShapeDtypeStruct((NNZ,), jnp.float32),
    mesh=_MESH,
    compiler_params=_SC_PARAMS,
    scratch_types=[
        pltpu.VMEM((STRIPE,), jnp.float32),
        pltpu.VMEM((STRIPE,), jnp.float32),
        pltpu.VMEM((STRIPE,), jnp.float32),
        pltpu.VMEM((NSEG,), jnp.float32),
        pltpu.VMEM((CHUNK,), jnp.int32),
        pltpu.VMEM((CHUNK,), jnp.int32),
        pltpu.VMEM((CHUNK,), jnp.float32),
        pltpu.VMEM((CHUNK,), jnp.float32),
        pltpu.VMEM_SHARED((NSEG,), jnp.float32),
        pltpu.SemaphoreType.DMA,
    ],
)
def _sc_normalize(ps_hbm, idx_hbm, ew_hbm, out_hbm,
                  p0_v, p1_v, is_v, inv_v, b_v, r_v, ew_v, o_v, shared, sem):
    cid = lax.axis_index("c")
    sid = lax.axis_index("s")
    base = (cid * 16 + sid) * CHUNK
    rbase = sid * STRIPE
    copies = [
        pltpu.async_copy(idx_hbm.at[pl.ds(base, CHUNK)], b_v, sem),
        pltpu.async_copy(idx_hbm.at[pl.ds(NNZ + base, CHUNK)], r_v, sem),
        pltpu.async_copy(ew_hbm.at[pl.ds(base, CHUNK)], ew_v, sem),
    ]
    pltpu.sync_copy(ps_hbm.at[0, pl.ds(rbase, STRIPE)], p0_v)
    pltpu.sync_copy(ps_hbm.at[1, pl.ds(rbase, STRIPE)], p1_v)

    ones16 = jnp.ones((16,), jnp.float32)
    for j in range(STRIPE // 16):
        sl = pl.ds(j * 16, 16)
        is_v[sl] = ones16 / (p0_v[sl] + p1_v[sl])
    pltpu.sync_copy(is_v, shared.at[pl.ds(rbase, STRIPE)])
    plsc.subcore_barrier()
    pltpu.sync_copy(shared, inv_v)
    for cp in copies:
        cp.wait()

    @plsc.parallel_loop(0, VECS, 1, unroll=4)
    def _(i):
        sl = pl.ds(i * 16, 16)
        seg = b_v[sl] * N1 + r_v[sl]
        g = plsc.load_gather(inv_v, [seg])
        o_v[sl] = ew_v[sl] * g

    pltpu.sync_copy(o_v, out_hbm.at[pl.ds(base, CHUNK)])


def kernel(t1, t2, H_indices, H_values, W1, b1, W2, b2, v):
    del H_values, b1, b2
    t1f = t1.reshape(B * N1, F1)
    t2f = t2.reshape(B * N2, F2)
    v2d = v.reshape(1, H_DIM)
    idx_flat = H_indices.reshape(3 * NNZ)
    a1, a2 = _matvec(t1f, t2f, W1, W2, v2d)
    ew, ps = _sc_logits(a1.reshape(NSEG), a2.reshape(NSEG), idx_flat)
    return _sc_normalize(ps, idx_flat, ew)


# speedup vs baseline: 253.5292x; 1.0018x over previous
import functools

import jax
import jax.numpy as jnp
from jax import lax
from jax.experimental import pallas as pl
from jax.experimental.pallas import tpu as pltpu
from jax.experimental.pallas import tpu_sc as plsc

B, N1, N2 = 4, 2048, 2048
F1 = F2 = 256
H_DIM = 256
NNZ = 262144

NSEG = B * N1
NW = 32
CHUNK = NNZ // NW
VECS = CHUNK // 16
STRIPE = NSEG // 16
MV_BLK = 4096


def _mv_body(t1_ref, t2_ref, w1_ref, w2_ref, v_ref, a1_ref, a2_ref):
    u1 = lax.dot_general(v_ref[...], w1_ref[...], (((1,), (0,)), ((), ())),
                         preferred_element_type=jnp.float32)
    u2 = lax.dot_general(v_ref[...], w2_ref[...], (((1,), (0,)), ((), ())),
                         preferred_element_type=jnp.float32)
    a1_ref[...] = jnp.sum(t1_ref[...] * u1, axis=1, keepdims=True)
    a2_ref[...] = jnp.sum(t2_ref[...] * u2, axis=1, keepdims=True)


_matvec = pl.pallas_call(
    _mv_body,
    grid=(B * N1 // MV_BLK,),
    in_specs=[
        pl.BlockSpec((MV_BLK, F1), lambda i: (i, 0)),
        pl.BlockSpec((MV_BLK, F2), lambda i: (i, 0)),
        pl.BlockSpec((H_DIM, F1), lambda i: (0, 0)),
        pl.BlockSpec((H_DIM, F2), lambda i: (0, 0)),
        pl.BlockSpec((1, H_DIM), lambda i: (0, 0)),
    ],
    out_specs=[pl.BlockSpec((MV_BLK, 1), lambda i: (i, 0)),
               pl.BlockSpec((MV_BLK, 1), lambda i: (i, 0))],
    out_shape=[jax.ShapeDtypeStruct((B * N1, 1), jnp.float32),
               jax.ShapeDtypeStruct((B * N2, 1), jnp.float32)],
)

_MESH = plsc.VectorSubcoreMesh(core_axis_name="c", subcore_axis_name="s")
_SC_PARAMS = pltpu.CompilerParams(needs_layout_passes=False)


@functools.partial(
    pl.kernel,
    out_type=(jax.ShapeDtypeStruct((NNZ,), jnp.float32),
              jax.ShapeDtypeStruct((2, NSEG), jnp.float32)),
    mesh=_MESH,
    compiler_params=_SC_PARAMS,
    scratch_types=[
        pltpu.VMEM((NSEG,), jnp.float32),
        pltpu.VMEM((NSEG,), jnp.float32),
        pltpu.VMEM((CHUNK,), jnp.int32),
        pltpu.VMEM((CHUNK,), jnp.int32),
        pltpu.VMEM((CHUNK,), jnp.int32),
        pltpu.VMEM((CHUNK,), jnp.float32),
        pltpu.VMEM((NSEG,), jnp.float32),
        pltpu.VMEM((STRIPE,), jnp.float32),
        pltpu.VMEM((16 * STRIPE,), jnp.float32),
        pltpu.VMEM_SHARED((16, NSEG), jnp.float32),
        pltpu.SemaphoreType.DMA,
    ],
)
def _sc_logits(a1_hbm, a2_hbm, idx_hbm, ew_hbm, ps_hbm,
               a1_v, a2_v, b_v, r_v, c_v, ew_v, tbl_v, acc_v, tmp_v, shared,
               sem):
    cid = lax.axis_index("c")
    sid = lax.axis_index("s")
    base = (cid * 16 + sid) * CHUNK
    rbase = sid * STRIPE
    copies = [
        pltpu.async_copy(a1_hbm, a1_v, sem),
        pltpu.async_copy(a2_hbm, a2_v, sem),
        pltpu.async_copy(idx_hbm.at[pl.ds(base, CHUNK)], b_v, sem),
        pltpu.async_copy(idx_hbm.at[pl.ds(NNZ + base, CHUNK)], r_v, sem),
        pltpu.async_copy(idx_hbm.at[pl.ds(2 * NNZ + base, CHUNK)], c_v, sem),
    ]

    zeros16 = jnp.zeros((16,), jnp.float32)

    def zero_body(i, carry):
        off = i * 64
        for u in range(4):
            tbl_v[pl.ds(off + u * 16, 16)] = zeros16
        return carry

    lax.fori_loop(0, NSEG // 64, zero_body, 0)
    for cp in copies:
        cp.wait()

    @plsc.parallel_loop(0, VECS, 1, unroll=4)
    def _(i):
        sl = pl.ds(i * 16, 16)
        b16 = b_v[sl]
        seg = b16 * N1 + r_v[sl]
        col = b16 * N2 + c_v[sl]
        e1 = plsc.load_gather(a1_v, [seg])
        e2 = plsc.load_gather(a2_v, [col])
        e = jnp.exp(e1 + e2)
        ew_v[sl] = e
        plsc.addupdate_scatter(tbl_v, [seg], e)

    out_cp = pltpu.async_copy(ew_v, ew_hbm.at[pl.ds(base, CHUNK)], sem)

    pltpu.sync_copy(tbl_v, shared.at[sid])
    plsc.subcore_barrier()
    red_cps = [pltpu.async_copy(shared.at[k, pl.ds(rbase, STRIPE)],
                                tmp_v.at[pl.ds(k * STRIPE, STRIPE)], sem)
               for k in range(16)]
    out_cp.wait()
    for cp in red_cps:
        cp.wait()
    for j in range(STRIPE // 16):
        sl = pl.ds(j * 16, 16)
        acc16 = tmp_v[pl.ds(j * 16, 16)]
        for k in range(1, 16):
            acc16 = acc16 + tmp_v[pl.ds(k * STRIPE + j * 16, 16)]
        acc_v[sl] = acc16
    pltpu.sync_copy(acc_v, ps_hbm.at[cid, pl.ds(rbase, STRIPE)])


@functools.partial(
    pl.kernel,
    out_type=jax.ShapeDtypeStruct((NNZ,), jnp.float32),
    mesh=_MESH,
    compiler_params=_SC_PARAMS,
    scratch_types=[
        pltpu.VMEM((STRIPE,), jnp.float32),
        pltpu.VMEM((STRIPE,), jnp.float32),
        pltpu.VMEM((STRIPE,), jnp.float32),
        pltpu.VMEM((NSEG,), jnp.float32),
        pltpu.VMEM((CHUNK,), jnp.int32),
        pltpu.VMEM((CHUNK,), jnp.int32),
        pltpu.VMEM((CHUNK,), jnp.float32),
        pltpu.VMEM((CHUNK,), jnp.float32),
        pltpu.VMEM_SHARED((NSEG,), jnp.float32),
        pltpu.SemaphoreType.DMA,
    ],
)
def _sc_normalize(ps_hbm, idx_hbm, ew_hbm, out_hbm,
                  p0_v, p1_v, is_v, inv_v, b_v, r_v, ew_v, o_v, shared, sem):
    cid = lax.axis_index("c")
    sid = lax.axis_index("s")
    base = (cid * 16 + sid) * CHUNK
    rbase = sid * STRIPE
    copies = [
        pltpu.async_copy(idx_hbm.at[pl.ds(base, CHUNK)], b_v, sem),
        pltpu.async_copy(idx_hbm.at[pl.ds(NNZ + base, CHUNK)], r_v, sem),
        pltpu.async_copy(ew_hbm.at[pl.ds(base, CHUNK)], ew_v, sem),
    ]
    pltpu.sync_copy(ps_hbm.at[0, pl.ds(rbase, STRIPE)], p0_v)
    pltpu.sync_copy(ps_hbm.at[1, pl.ds(rbase, STRIPE)], p1_v)

    ones16 = jnp.ones((16,), jnp.float32)
    for j in range(STRIPE // 16):
        sl = pl.ds(j * 16, 16)
        is_v[sl] = ones16 / (p0_v[sl] + p1_v[sl])
    pltpu.sync_copy(is_v, shared.at[pl.ds(rbase, STRIPE)])
    plsc.subcore_barrier()
    pltpu.sync_copy(shared, inv_v)
    for cp in copies:
        cp.wait()

    @plsc.parallel_loop(0, VECS, 1, unroll=4)
    def _(i):
        sl = pl.ds(i * 16, 16)
        seg = b_v[sl] * N1 + r_v[sl]
        g = plsc.load_gather(inv_v, [seg])
        o_v[sl] = ew_v[sl] * g

    pltpu.sync_copy(o_v, out_hbm.at[pl.ds(base, CHUNK)])


def kernel(t1, t2, H_indices, H_values, W1, b1, W2, b2, v):
    del H_values, b1, b2
    t1f = t1.reshape(B * N1, F1)
    t2f = t2.reshape(B * N2, F2)
    v2d = v.reshape(1, H_DIM)
    idx_flat = H_indices.reshape(3 * NNZ)
    a1, a2 = _matvec(t1f, t2f, W1, W2, v2d)
    ew, ps = _sc_logits(a1.reshape(NSEG), a2.reshape(NSEG), idx_flat)
    return _sc_normalize(ps, idx_flat, ew)
